# baseline (speedup 1.0000x reference)
import functools

import jax
import jax.numpy as jnp
from jax import lax
from jax.experimental import pallas as pl
from jax.experimental.pallas import tpu as pltpu
from jax.experimental.pallas import tpu_sc as plsc

NUM_SP = 1000
B, H, W, C = 4, 384, 384, 96
CP = 128
NPIX = B * H * W
NSEG = B * NUM_SP
NC, NS = 2, 16
SPLIT = 2
B_CALL = B // SPLIT
PIX_CALL = NPIX // SPLIT
PIX_CORE = PIX_CALL // NC
PIX_TILE = PIX_CORE // NS
CH = 128
NCHUNK = PIX_TILE // CH
SUP = 512
NSUPER = PIX_TILE // SUP
SCAT = SUP // CH
SEGP = 1024
SEG_CHUNK = SEGP // NS
HBLK = 32


def _transpose_body(in_ref, out_ref):
  x = in_ref[0]
  xt = jnp.swapaxes(x, 1, 2).reshape(HBLK * W, C)
  out_ref[:, pl.ds(0, C)] = xt


def _tc_transpose_pad(fm_t, half):
  return pl.pallas_call(
      _transpose_body,
      grid=(B_CALL, H // HBLK),
      in_specs=[pl.BlockSpec((1, HBLK, C, W),
                             lambda b, h: (b + half * B_CALL, h, 0, 0))],
      out_specs=pl.BlockSpec((HBLK * W, CP),
                             lambda b, h: (b * (H // HBLK) + h, 0)),
      out_shape=jax.ShapeDtypeStruct((PIX_CALL, CP), jnp.float32),
      name=f"transpose_pad_h{half}",
  )(fm_t)


def _sc_segment_mean(featp, sp2d, z2, z1, ones, half):
  mesh = plsc.VectorSubcoreMesh(core_axis_name="c", subcore_axis_name="s")

  @functools.partial(
      pl.kernel,
      mesh=mesh,
      out_type=jax.ShapeDtypeStruct((NC * SEGP, C), jnp.float32),
      scratch_types=[
          pltpu.VMEM((NCHUNK, CH), jnp.int32),
          pltpu.VMEM((2, SUP, C), jnp.float32),
          pltpu.VMEM((CH,), jnp.float32),
          pltpu.VMEM((SEG_CHUNK + 16,), jnp.float32),
          pltpu.VMEM_SHARED((SEGP, C), jnp.float32),
          pltpu.VMEM_SHARED((SEGP,), jnp.float32),
          pltpu.SemaphoreType.DMA,
          pltpu.SemaphoreType.DMA,
          pltpu.SemaphoreType.DMA,
          pltpu.SemaphoreType.DMA,
      ],
      compiler_params=pltpu.CompilerParams(use_tc_tiling_on_sc=False),
      name=f"segment_mean_h{half}",
  )
  def k(feat_hbm, sp_hbm, z2_hbm, z1_hbm, ones_hbm, means_out,
        idx_v, feat_v, ones_v, cnt_v, sums_sh, cnts_sh,
        lsem0, lsem1, ssem0, ssem1):
    c = lax.axis_index("c")
    s = lax.axis_index("s")
    pix0 = pl.multiple_of(c * PIX_CORE + s * PIX_TILE, PIX_TILE)
    row0 = pl.multiple_of((half * PIX_CALL + pix0) // CH, NCHUNK)
    seg0 = pl.multiple_of(s * SEG_CHUNK, SEG_CHUNK)

    pltpu.async_copy(feat_hbm.at[pl.ds(pix0, SUP), pl.ds(0, C)],
                     feat_v.at[0], lsem0)
    pltpu.async_copy(
        feat_hbm.at[pl.ds(pl.multiple_of(pix0 + SUP, SUP), SUP), pl.ds(0, C)],
        feat_v.at[1], lsem1)

    pltpu.sync_copy(z2_hbm, sums_sh.at[pl.ds(seg0, SEG_CHUNK)])
    pltpu.sync_copy(z1_hbm, cnts_sh.at[pl.ds(seg0, SEG_CHUNK)])

    pltpu.sync_copy(sp_hbm.at[pl.ds(row0, NCHUNK)], idx_v)
    pltpu.sync_copy(ones_hbm, ones_v)
    plsc.subcore_barrier()

    lsems = (lsem0, lsem1)
    ssems = (ssem0, ssem1)

    def ring(g, carry):
      for b in range(2):
        kk = 2 * g + b
        pltpu.make_async_copy(
            feat_hbm.at[pl.ds(0, SUP), pl.ds(0, C)], feat_v.at[b],
            lsems[b]).wait()
        handles = []
        for j in range(SCAT):
          idxrow = idx_v.at[kk * SCAT + j]
          handles.append(pltpu.async_copy(
              feat_v.at[b, pl.ds(j * CH, CH)], sums_sh.at[idxrow],
              ssems[b], add=True))
          handles.append(pltpu.async_copy(
              ones_v, cnts_sh.at[idxrow], ssems[b], add=True))
        for h in handles:
          h.wait()

        @pl.when(kk + 2 < NSUPER)
        def _():
          p = pl.multiple_of(pix0 + (kk + 2) * SUP, SUP)
          pltpu.async_copy(feat_hbm.at[pl.ds(p, SUP), pl.ds(0, C)],
                           feat_v.at[b], lsems[b])

      return carry

    lax.fori_loop(0, NSUPER // 2, ring, 0)
    plsc.subcore_barrier()

    pltpu.sync_copy(cnts_sh.at[pl.ds(seg0, SEG_CHUNK)],
                    cnt_v.at[pl.ds(0, SEG_CHUNK)])
    pltpu.sync_copy(sums_sh.at[pl.ds(seg0, SEG_CHUNK)],
                    feat_v.at[0, pl.ds(0, SEG_CHUNK)])
    one_vec = jnp.ones((16,), jnp.float32)
    for j in range(SEG_CHUNK // 16):
      cnt_v[pl.ds(j * 16, 16)] = one_vec / cnt_v[pl.ds(j * 16, 16)]

    def div_row(r, carry):
      inv = jnp.full((16,), cnt_v[pl.ds(r, 16)][0], jnp.float32)
      for j in range(C // 16):
        feat_v[0, r, pl.ds(j * 16, 16)] = (
            feat_v[0, r, pl.ds(j * 16, 16)] * inv)
      return carry

    lax.fori_loop(0, SEG_CHUNK, div_row, 0)
    out0 = pl.multiple_of(c * SEGP + seg0, SEG_CHUNK)
    pltpu.sync_copy(feat_v.at[0, pl.ds(0, SEG_CHUNK)],
                    means_out.at[pl.ds(out0, SEG_CHUNK)])

  return k(featp, sp2d, z2, z1, ones)


def kernel(feature_map, superpixel_map):
  fm_t = feature_map.transpose(0, 1, 3, 2)
  sp2d = superpixel_map.astype(jnp.int32).reshape(NPIX // CH, CH)
  z2 = jnp.zeros((SEG_CHUNK, C), jnp.float32)
  z1 = jnp.zeros((SEG_CHUNK,), jnp.float32)
  ones = jnp.ones((CH,), jnp.float32)
  parts = []
  for half in range(SPLIT):
    featp = _tc_transpose_pad(fm_t, half)
    mp = _sc_segment_mean(featp, sp2d, z2, z1, ones, half)
    parts.append(mp[:NUM_SP])
    parts.append(mp[SEGP:SEGP + NUM_SP])
  return jnp.concatenate(parts, axis=0).reshape(B, NUM_SP, C)


# speedup vs baseline: 1.0077x; 1.0077x over previous
import functools

import jax
import jax.numpy as jnp
from jax import lax
from jax.experimental import pallas as pl
from jax.experimental.pallas import tpu as pltpu
from jax.experimental.pallas import tpu_sc as plsc

NUM_SP = 1000
B, H, W, C = 4, 384, 384, 96
CP = 128
NPIX = B * H * W
NSEG = B * NUM_SP
NC, NS = 2, 16
SPLIT = 2
B_CALL = B // SPLIT
PIX_CALL = NPIX // SPLIT
PIX_CORE = PIX_CALL // NC
PIX_TILE = PIX_CORE // NS
CH = 128
NCHUNK = PIX_TILE // CH
SUP = 256
NSUPER = PIX_TILE // SUP
SCAT = SUP // CH
SEGP = 1024
SEG_CHUNK = SEGP // NS
HBLK = 32


def _transpose_body(in_ref, out_ref):
  x = in_ref[0]
  xt = jnp.swapaxes(x, 1, 2).reshape(HBLK * W, C)
  out_ref[:, pl.ds(0, C)] = xt


def _tc_transpose_pad(fm_t, half):
  return pl.pallas_call(
      _transpose_body,
      grid=(B_CALL, H // HBLK),
      in_specs=[pl.BlockSpec((1, HBLK, C, W),
                             lambda b, h: (b + half * B_CALL, h, 0, 0))],
      out_specs=pl.BlockSpec((HBLK * W, CP),
                             lambda b, h: (b * (H // HBLK) + h, 0)),
      out_shape=jax.ShapeDtypeStruct((PIX_CALL, CP), jnp.float32),
      name=f"transpose_pad_h{half}",
  )(fm_t)


def _sc_segment_mean(featp, sp2d, z2, z1, ones, half):
  mesh = plsc.VectorSubcoreMesh(core_axis_name="c", subcore_axis_name="s")

  @functools.partial(
      pl.kernel,
      mesh=mesh,
      out_type=jax.ShapeDtypeStruct((NC * SEGP, C), jnp.float32),
      scratch_types=[
          pltpu.VMEM((NCHUNK, CH), jnp.int32),
          pltpu.VMEM((2, SUP, C), jnp.float32),
          pltpu.VMEM((CH,), jnp.float32),
          pltpu.VMEM((SEG_CHUNK + 16,), jnp.float32),
          pltpu.VMEM_SHARED((SEGP, C), jnp.float32),
          pltpu.VMEM_SHARED((SEGP,), jnp.float32),
          pltpu.SemaphoreType.DMA,
          pltpu.SemaphoreType.DMA,
          pltpu.SemaphoreType.DMA,
          pltpu.SemaphoreType.DMA,
      ],
      compiler_params=pltpu.CompilerParams(use_tc_tiling_on_sc=False),
      name=f"segment_mean_h{half}",
  )
  def k(feat_hbm, sp_hbm, z2_hbm, z1_hbm, ones_hbm, means_out,
        idx_v, feat_v, ones_v, cnt_v, sums_sh, cnts_sh,
        lsem0, lsem1, ssem0, ssem1):
    c = lax.axis_index("c")
    s = lax.axis_index("s")
    pix0 = pl.multiple_of(c * PIX_CORE + s * PIX_TILE, PIX_TILE)
    row0 = pl.multiple_of((half * PIX_CALL + pix0) // CH, NCHUNK)
    seg0 = pl.multiple_of(s * SEG_CHUNK, SEG_CHUNK)

    pltpu.async_copy(feat_hbm.at[pl.ds(pix0, SUP), pl.ds(0, C)],
                     feat_v.at[0], lsem0)
    pltpu.async_copy(
        feat_hbm.at[pl.ds(pl.multiple_of(pix0 + SUP, SUP), SUP), pl.ds(0, C)],
        feat_v.at[1], lsem1)

    pltpu.sync_copy(z2_hbm, sums_sh.at[pl.ds(seg0, SEG_CHUNK)])
    pltpu.sync_copy(z1_hbm, cnts_sh.at[pl.ds(seg0, SEG_CHUNK)])

    pltpu.sync_copy(sp_hbm.at[pl.ds(row0, NCHUNK)], idx_v)
    pltpu.sync_copy(ones_hbm, ones_v)
    plsc.subcore_barrier()

    lsems = (lsem0, lsem1)
    ssems = (ssem0, ssem1)

    def ring(g, carry):
      for b in range(2):
        kk = 2 * g + b
        pltpu.make_async_copy(
            feat_hbm.at[pl.ds(0, SUP), pl.ds(0, C)], feat_v.at[b],
            lsems[b]).wait()
        handles = []
        for j in range(SCAT):
          idxrow = idx_v.at[kk * SCAT + j]
          handles.append(pltpu.async_copy(
              feat_v.at[b, pl.ds(j * CH, CH)], sums_sh.at[idxrow],
              ssems[b], add=True))
          handles.append(pltpu.async_copy(
              ones_v, cnts_sh.at[idxrow], ssems[b], add=True))
        for h in handles:
          h.wait()

        @pl.when(kk + 2 < NSUPER)
        def _():
          p = pl.multiple_of(pix0 + (kk + 2) * SUP, SUP)
          pltpu.async_copy(feat_hbm.at[pl.ds(p, SUP), pl.ds(0, C)],
                           feat_v.at[b], lsems[b])

      return carry

    lax.fori_loop(0, NSUPER // 2, ring, 0)
    plsc.subcore_barrier()

    pltpu.sync_copy(cnts_sh.at[pl.ds(seg0, SEG_CHUNK)],
                    cnt_v.at[pl.ds(0, SEG_CHUNK)])
    pltpu.sync_copy(sums_sh.at[pl.ds(seg0, SEG_CHUNK)],
                    feat_v.at[0, pl.ds(0, SEG_CHUNK)])
    one_vec = jnp.ones((16,), jnp.float32)
    for j in range(SEG_CHUNK // 16):
      cnt_v[pl.ds(j * 16, 16)] = one_vec / cnt_v[pl.ds(j * 16, 16)]

    def div_row(r, carry):
      inv = jnp.full((16,), cnt_v[pl.ds(r, 16)][0], jnp.float32)
      for j in range(C // 16):
        feat_v[0, r, pl.ds(j * 16, 16)] = (
            feat_v[0, r, pl.ds(j * 16, 16)] * inv)
      return carry

    lax.fori_loop(0, SEG_CHUNK, div_row, 0)
    out0 = pl.multiple_of(c * SEGP + seg0, SEG_CHUNK)
    pltpu.sync_copy(feat_v.at[0, pl.ds(0, SEG_CHUNK)],
                    means_out.at[pl.ds(out0, SEG_CHUNK)])

  return k(featp, sp2d, z2, z1, ones)


def kernel(feature_map, superpixel_map):
  fm_t = feature_map.transpose(0, 1, 3, 2)
  sp2d = superpixel_map.astype(jnp.int32).reshape(NPIX // CH, CH)
  z2 = jnp.zeros((SEG_CHUNK, C), jnp.float32)
  z1 = jnp.zeros((SEG_CHUNK,), jnp.float32)
  ones = jnp.ones((CH,), jnp.float32)
  parts = []
  for half in range(SPLIT):
    featp = _tc_transpose_pad(fm_t, half)
    mp = _sc_segment_mean(featp, sp2d, z2, z1, ones, half)
    parts.append(mp[:NUM_SP])
    parts.append(mp[SEGP:SEGP + NUM_SP])
  return jnp.concatenate(parts, axis=0).reshape(B, NUM_SP, C)
